# baseline (speedup 1.0000x reference)
import functools

import jax
import jax.numpy as jnp
from jax import lax
from jax.experimental import pallas as pl
from jax.experimental.pallas import tpu as pltpu
from jax.experimental.pallas import tpu_sc as plsc

NC = 2
NS = 16
L = 16
S = 32
D = 128
C = 4
HP = 5120
BP = 2 * HP
BPW = HP // NS
NCH = BPW // C
NVREG = D // L


@functools.partial(
    pl.kernel,
    out_type=jax.ShapeDtypeStruct((HP, D), jnp.float32),
    mesh=plsc.VectorSubcoreMesh(
        core_axis_name="c", subcore_axis_name="s",
        num_cores=NC, num_subcores=NS),
    scratch_types=[
        pltpu.VMEM((NCH, C * S), jnp.int32),
        pltpu.VMEM((C * S, D), jnp.float32),
        pltpu.VMEM((C * S, D), jnp.float32),
        pltpu.VMEM((BPW, D), jnp.float32),
        pltpu.SemaphoreType.DMA,
        pltpu.SemaphoreType.DMA,
    ],
)
def _mean_agg_half(idx_hbm, table_hbm, out_hbm, idx_v, buf0, buf1, out_v,
                   sem0, sem1):
    cid = lax.axis_index("c")
    sid = lax.axis_index("s")

    bufs = (buf0, buf1)
    sems = (sem0, sem1)

    @pl.when(cid == 0)
    def _():
        pltpu.sync_copy(idx_hbm.at[pl.ds(sid * NCH, NCH)], idx_v)
        pltpu.async_copy(table_hbm.at[idx_v.at[0]], buf0, sem0)
        pltpu.async_copy(table_hbm.at[idx_v.at[1]], buf1, sem1)

        def compute(c, buf):
            for r in range(C):
                def body(s_, carry):
                    row = r * S + s_
                    return tuple(
                        a + buf[row, pl.ds(v * L, L)]
                        for v, a in enumerate(carry))
                acc = lax.fori_loop(
                    0, S, body,
                    tuple(jnp.zeros((L,), jnp.float32)
                          for _ in range(NVREG)))
                orow = c * C + r
                for v in range(NVREG):
                    out_v[orow, pl.ds(v * L, L)] = acc[v] * (1.0 / S)

        def outer(g, carry):
            for b in range(2):
                c = g * 2 + b
                pltpu.make_async_copy(
                    table_hbm.at[idx_v.at[0]], bufs[b], sems[b]).wait()
                compute(c, bufs[b])

                @pl.when(c + 2 < NCH)
                def _():
                    pltpu.async_copy(
                        table_hbm.at[idx_v.at[c + 2]], bufs[b], sems[b])
            return carry

        lax.fori_loop(0, NCH // 2, outer, 0)
        pltpu.sync_copy(out_v, out_hbm.at[pl.ds(sid * BPW, BPW)])


def kernel(nodes, to_neighs, features_table):
    del nodes
    b = to_neighs.shape[0]
    idx = jnp.pad(to_neighs.astype(jnp.int32), ((0, BP - b), (0, 0)))
    idx2d = idx.reshape(BP * S // (C * S), C * S)
    half_rows = HP * S // (C * S)
    out0 = _mean_agg_half(idx2d[:half_rows], features_table)
    out1 = _mean_agg_half(idx2d[half_rows:], features_table)
    out = jnp.concatenate([out0, out1], axis=0)
    return out[:b]


# speedup vs baseline: 1.0844x; 1.0844x over previous
import functools

import jax
import jax.numpy as jnp
from jax import lax
from jax.experimental import pallas as pl
from jax.experimental.pallas import tpu as pltpu
from jax.experimental.pallas import tpu_sc as plsc

NC = 2
NS = 16
L = 16
S = 32
D = 128
C = 4
NBUF = 4
BP = 10240
BPW = BP // NS
NCH = BPW // C
NVREG = D // L


@functools.partial(
    pl.kernel,
    out_type=jax.ShapeDtypeStruct((BP, D), jnp.float32),
    mesh=plsc.VectorSubcoreMesh(
        core_axis_name="c", subcore_axis_name="s",
        num_cores=NC, num_subcores=NS),
    scratch_types=(
        [pltpu.VMEM((NCH, C * S), jnp.int32)]
        + [pltpu.VMEM((C * S, D), jnp.float32) for _ in range(NBUF)]
        + [pltpu.VMEM((C, D), jnp.float32) for _ in range(2)]
        + [pltpu.SemaphoreType.DMA for _ in range(NBUF + 2)]
    ),
)
def _mean_agg(idx_hbm, table_hbm, out_hbm, idx_v, *rest):
    bufs = rest[:NBUF]
    orings = rest[NBUF:NBUF + 2]
    sems = rest[NBUF + 2:2 * NBUF + 2]
    osems = rest[2 * NBUF + 2:]
    cid = lax.axis_index("c")
    sid = lax.axis_index("s")

    @pl.when(cid == 0)
    def _():
        out_base = sid * BPW
        pltpu.sync_copy(idx_hbm.at[pl.ds(sid * NCH, NCH)], idx_v)
        for b in range(NBUF):
            pltpu.async_copy(table_hbm.at[idx_v.at[b]], bufs[b], sems[b])

        def compute(c, buf, oring):
            for r in range(C):
                def body(s_, carry):
                    row = r * S + s_
                    return tuple(
                        a + buf[row, pl.ds(v * L, L)]
                        for v, a in enumerate(carry))
                acc = lax.fori_loop(
                    0, S, body,
                    tuple(jnp.zeros((L,), jnp.float32)
                          for _ in range(NVREG)))
                for v in range(NVREG):
                    oring[r, pl.ds(v * L, L)] = acc[v] * (1.0 / S)

        def outer(g, carry):
            for b in range(NBUF):
                c = g * NBUF + b
                ob = b % 2
                pltpu.make_async_copy(
                    table_hbm.at[idx_v.at[0]], bufs[b], sems[b]).wait()

                @pl.when(c >= 2)
                def _():
                    pltpu.make_async_copy(
                        orings[ob], out_hbm.at[pl.ds(0, C)],
                        osems[ob]).wait()

                compute(c, bufs[b], orings[ob])
                pltpu.async_copy(
                    orings[ob], out_hbm.at[pl.ds(out_base + c * C, C)],
                    osems[ob])

                @pl.when(c + NBUF < NCH)
                def _():
                    pltpu.async_copy(
                        table_hbm.at[idx_v.at[c + NBUF]], bufs[b], sems[b])
            return carry

        lax.fori_loop(0, NCH // NBUF, outer, 0)
        for ob in range(2):
            pltpu.make_async_copy(
                orings[ob], out_hbm.at[pl.ds(0, C)], osems[ob]).wait()


def kernel(nodes, to_neighs, features_table):
    del nodes
    b = to_neighs.shape[0]
    idx = jnp.pad(to_neighs.astype(jnp.int32), ((0, BP - b), (0, 0)))
    idx2d = idx.reshape(BP * S // (C * S), C * S)
    out = _mean_agg(idx2d, features_table)
    return out[:b]


# speedup vs baseline: 1.0959x; 1.0106x over previous
import functools

import jax
import jax.numpy as jnp
from jax import lax
from jax.experimental import pallas as pl
from jax.experimental.pallas import tpu as pltpu
from jax.experimental.pallas import tpu_sc as plsc

NC = 2
NS = 16
L = 16
S = 32
D = 128
C = 4
NBUF = 4
BP = 10240
BPW0 = 512
BPW1 = 128
NCH0 = BPW0 // C
NCH1 = BPW1 // C
NVREG = D // L


@functools.partial(
    pl.kernel,
    out_type=jax.ShapeDtypeStruct((BP, D), jnp.float32),
    mesh=plsc.VectorSubcoreMesh(
        core_axis_name="c", subcore_axis_name="s",
        num_cores=NC, num_subcores=NS),
    scratch_types=(
        [pltpu.VMEM((NCH0, C * S), jnp.int32)]
        + [pltpu.VMEM((C * S, D), jnp.float32) for _ in range(NBUF)]
        + [pltpu.VMEM((C, D), jnp.float32) for _ in range(2)]
        + [pltpu.SemaphoreType.DMA for _ in range(NBUF + 2)]
    ),
)
def _mean_agg(idx_hbm, table_hbm, out_hbm, idx_v, *rest):
    bufs = rest[:NBUF]
    orings = rest[NBUF:NBUF + 2]
    sems = rest[NBUF + 2:2 * NBUF + 2]
    osems = rest[2 * NBUF + 2:]
    cid = lax.axis_index("c")
    sid = lax.axis_index("s")

    def compute(c, buf, oring):
        for r in range(C):
            def body(s_, carry):
                row = r * S + s_
                return tuple(
                    a + buf[row, pl.ds(v * L, L)]
                    for v, a in enumerate(carry))
            acc = lax.fori_loop(
                0, S, body,
                tuple(jnp.zeros((L,), jnp.float32)
                      for _ in range(NVREG)))
            for v in range(NVREG):
                oring[r, pl.ds(v * L, L)] = acc[v] * (1.0 / S)

    def run(nch, out_base, idx_base):
        pltpu.sync_copy(idx_hbm.at[pl.ds(idx_base, nch)],
                        idx_v.at[pl.ds(0, nch)])
        for b in range(NBUF):
            pltpu.async_copy(table_hbm.at[idx_v.at[b]], bufs[b], sems[b])

        def outer(g, carry):
            for b in range(NBUF):
                c = g * NBUF + b
                ob = b % 2
                pltpu.make_async_copy(
                    table_hbm.at[idx_v.at[0]], bufs[b], sems[b]).wait()

                @pl.when(c >= 2)
                def _():
                    pltpu.make_async_copy(
                        orings[ob], out_hbm.at[pl.ds(0, C)],
                        osems[ob]).wait()

                compute(c, bufs[b], orings[ob])
                pltpu.async_copy(
                    orings[ob], out_hbm.at[pl.ds(out_base + c * C, C)],
                    osems[ob])

                @pl.when(c + NBUF < nch)
                def _():
                    pltpu.async_copy(
                        table_hbm.at[idx_v.at[c + NBUF]], bufs[b], sems[b])
            return carry

        lax.fori_loop(0, nch // NBUF, outer, 0)
        for ob in range(2):
            pltpu.make_async_copy(
                orings[ob], out_hbm.at[pl.ds(0, C)], osems[ob]).wait()

    @pl.when(cid == 0)
    def _():
        run(NCH0, sid * BPW0, sid * NCH0)

    @pl.when(cid == 1)
    def _():
        run(NCH1, NS * BPW0 + sid * BPW1, NS * NCH0 + sid * NCH1)


def kernel(nodes, to_neighs, features_table):
    del nodes
    b = to_neighs.shape[0]
    idx = jnp.pad(to_neighs.astype(jnp.int32), ((0, BP - b), (0, 0)))
    idx2d = idx.reshape(BP * S // (C * S), C * S)
    out = _mean_agg(idx2d, features_table)
    return out[:b]


# speedup vs baseline: 1.1252x; 1.0268x over previous
import functools

import jax
import jax.numpy as jnp
from jax import lax
from jax.experimental import pallas as pl
from jax.experimental.pallas import tpu as pltpu
from jax.experimental.pallas import tpu_sc as plsc

NC = 2
NS = 16
L = 16
S = 32
D = 128
C = 4
BP = 10240
NVREG = D // L
BPW0 = 512
BPW1 = 128
NCH0 = BPW0 // C
NCH1 = BPW1 // C


@functools.partial(
    pl.kernel,
    out_type=jax.ShapeDtypeStruct((BP, D), jnp.float32),
    mesh=plsc.VectorSubcoreMesh(
        core_axis_name="c", subcore_axis_name="s",
        num_cores=NC, num_subcores=NS),
    scratch_types=[
        pltpu.VMEM((NCH0, C * S), jnp.int32),
        pltpu.VMEM((C * S, D), jnp.float32),
        pltpu.VMEM((C * S, D), jnp.float32),
        pltpu.VMEM((BPW0, D), jnp.float32),
        pltpu.SemaphoreType.DMA,
        pltpu.SemaphoreType.DMA,
    ],
)
def _mean_agg(idx_hbm, table_hbm, out_hbm, idx_v, buf0, buf1, out_v,
              sem0, sem1):
    cid = lax.axis_index("c")
    sid = lax.axis_index("s")

    bufs = (buf0, buf1)
    sems = (sem0, sem1)

    def compute(c, buf):
        for r in range(C):
            def body(s_, carry):
                row = r * S + s_
                return tuple(
                    a + buf[row, pl.ds(v * L, L)]
                    for v, a in enumerate(carry))
            acc = lax.fori_loop(
                0, S, body,
                tuple(jnp.zeros((L,), jnp.float32) for _ in range(NVREG)))
            orow = c * C + r
            for v in range(NVREG):
                out_v[orow, pl.ds(v * L, L)] = acc[v] * (1.0 / S)

    def run(nch, out_base, idx_base):
        pltpu.sync_copy(idx_hbm.at[pl.ds(idx_base, nch)],
                        idx_v.at[pl.ds(0, nch)])
        pltpu.async_copy(table_hbm.at[idx_v.at[0]], buf0, sem0)
        pltpu.async_copy(table_hbm.at[idx_v.at[1]], buf1, sem1)

        def outer(g, carry):
            for b in range(2):
                c = g * 2 + b
                pltpu.make_async_copy(
                    table_hbm.at[idx_v.at[0]], bufs[b], sems[b]).wait()
                compute(c, bufs[b])

                @pl.when(c + 2 < nch)
                def _():
                    pltpu.async_copy(
                        table_hbm.at[idx_v.at[c + 2]], bufs[b], sems[b])
            return carry

        lax.fori_loop(0, nch // 2, outer, 0)
        pltpu.sync_copy(out_v.at[pl.ds(0, nch * C)],
                        out_hbm.at[pl.ds(out_base, nch * C)])

    @pl.when(cid == 0)
    def _():
        run(NCH0, sid * BPW0, sid * NCH0)

    @pl.when(cid == 1)
    def _():
        run(NCH1, NS * BPW0 + sid * BPW1, NS * NCH0 + sid * NCH1)


def kernel(nodes, to_neighs, features_table):
    del nodes
    b = to_neighs.shape[0]
    idx = jnp.pad(to_neighs.astype(jnp.int32), ((0, BP - b), (0, 0)))
    idx2d = idx.reshape(BP * S // (C * S), C * S)
    out = _mean_agg(idx2d, features_table)
    return out[:b]


# speedup vs baseline: 1.1482x; 1.0204x over previous
import functools

import jax
import jax.numpy as jnp
from jax import lax
from jax.experimental import pallas as pl
from jax.experimental.pallas import tpu as pltpu
from jax.experimental.pallas import tpu_sc as plsc

NC = 2
NS = 16
L = 16
S = 32
D = 128
C = 4
BP = 10240
NVREG = D // L
BPW0 = 544
BPW1 = 96
NCH0 = BPW0 // C
NCH1 = BPW1 // C


@functools.partial(
    pl.kernel,
    out_type=jax.ShapeDtypeStruct((BP, D), jnp.float32),
    mesh=plsc.VectorSubcoreMesh(
        core_axis_name="c", subcore_axis_name="s",
        num_cores=NC, num_subcores=NS),
    scratch_types=[
        pltpu.VMEM((NCH0, C * S), jnp.int32),
        pltpu.VMEM((C * S, D), jnp.float32),
        pltpu.VMEM((C * S, D), jnp.float32),
        pltpu.VMEM((BPW0, D), jnp.float32),
        pltpu.SemaphoreType.DMA,
        pltpu.SemaphoreType.DMA,
    ],
)
def _mean_agg(idx_hbm, table_hbm, out_hbm, idx_v, buf0, buf1, out_v,
              sem0, sem1):
    cid = lax.axis_index("c")
    sid = lax.axis_index("s")

    bufs = (buf0, buf1)
    sems = (sem0, sem1)

    def compute(c, buf):
        for r in range(C):
            def body(s_, carry):
                row = r * S + s_
                return tuple(
                    a + buf[row, pl.ds(v * L, L)]
                    for v, a in enumerate(carry))
            acc = lax.fori_loop(
                0, S, body,
                tuple(jnp.zeros((L,), jnp.float32) for _ in range(NVREG)))
            orow = c * C + r
            for v in range(NVREG):
                out_v[orow, pl.ds(v * L, L)] = acc[v] * (1.0 / S)

    def run(nch, out_base, idx_base):
        pltpu.sync_copy(idx_hbm.at[pl.ds(idx_base, nch)],
                        idx_v.at[pl.ds(0, nch)])
        pltpu.async_copy(table_hbm.at[idx_v.at[0]], buf0, sem0)
        pltpu.async_copy(table_hbm.at[idx_v.at[1]], buf1, sem1)

        def outer(g, carry):
            for b in range(2):
                c = g * 2 + b
                pltpu.make_async_copy(
                    table_hbm.at[idx_v.at[0]], bufs[b], sems[b]).wait()
                compute(c, bufs[b])

                @pl.when(c + 2 < nch)
                def _():
                    pltpu.async_copy(
                        table_hbm.at[idx_v.at[c + 2]], bufs[b], sems[b])
            return carry

        lax.fori_loop(0, nch // 2, outer, 0)
        pltpu.sync_copy(out_v.at[pl.ds(0, nch * C)],
                        out_hbm.at[pl.ds(out_base, nch * C)])

    @pl.when(cid == 0)
    def _():
        run(NCH0, sid * BPW0, sid * NCH0)

    @pl.when(cid == 1)
    def _():
        run(NCH1, NS * BPW0 + sid * BPW1, NS * NCH0 + sid * NCH1)


def kernel(nodes, to_neighs, features_table):
    del nodes
    b = to_neighs.shape[0]
    idx = jnp.pad(to_neighs.astype(jnp.int32), ((0, BP - b), (0, 0)))
    idx2d = idx.reshape(BP * S // (C * S), C * S)
    out = _mean_agg(idx2d, features_table)
    return out[:b]


# speedup vs baseline: 1.2820x; 1.1166x over previous
import functools

import jax
import jax.numpy as jnp
from jax import lax
from jax.experimental import pallas as pl
from jax.experimental.pallas import tpu as pltpu
from jax.experimental.pallas import tpu_sc as plsc

NC = 2
NS = 16
L = 16
S = 32
D = 128
C = 4
BP = 10240
NVREG = D // L
BPW0 = 576
BPW1 = 64
NCH0 = BPW0 // C
NCH1 = BPW1 // C


@functools.partial(
    pl.kernel,
    out_type=jax.ShapeDtypeStruct((BP, D), jnp.float32),
    mesh=plsc.VectorSubcoreMesh(
        core_axis_name="c", subcore_axis_name="s",
        num_cores=NC, num_subcores=NS),
    scratch_types=[
        pltpu.VMEM((NCH0, C * S), jnp.int32),
        pltpu.VMEM((C * S, D), jnp.float32),
        pltpu.VMEM((C * S, D), jnp.float32),
        pltpu.VMEM((BPW0, D), jnp.float32),
        pltpu.SemaphoreType.DMA,
        pltpu.SemaphoreType.DMA,
    ],
)
def _mean_agg(idx_hbm, table_hbm, out_hbm, idx_v, buf0, buf1, out_v,
              sem0, sem1):
    cid = lax.axis_index("c")
    sid = lax.axis_index("s")

    bufs = (buf0, buf1)
    sems = (sem0, sem1)

    def compute(c, buf):
        for r in range(C):
            def body(s_, carry):
                row = r * S + s_
                return tuple(
                    a + buf[row, pl.ds(v * L, L)]
                    for v, a in enumerate(carry))
            acc = lax.fori_loop(
                0, S, body,
                tuple(jnp.zeros((L,), jnp.float32) for _ in range(NVREG)))
            orow = c * C + r
            for v in range(NVREG):
                out_v[orow, pl.ds(v * L, L)] = acc[v] * (1.0 / S)

    def run(nch, out_base, idx_base):
        pltpu.sync_copy(idx_hbm.at[pl.ds(idx_base, nch)],
                        idx_v.at[pl.ds(0, nch)])
        pltpu.async_copy(table_hbm.at[idx_v.at[0]], buf0, sem0)
        pltpu.async_copy(table_hbm.at[idx_v.at[1]], buf1, sem1)

        def outer(g, carry):
            for b in range(2):
                c = g * 2 + b
                pltpu.make_async_copy(
                    table_hbm.at[idx_v.at[0]], bufs[b], sems[b]).wait()
                compute(c, bufs[b])

                @pl.when(c + 2 < nch)
                def _():
                    pltpu.async_copy(
                        table_hbm.at[idx_v.at[c + 2]], bufs[b], sems[b])
            return carry

        lax.fori_loop(0, nch // 2, outer, 0)
        pltpu.sync_copy(out_v.at[pl.ds(0, nch * C)],
                        out_hbm.at[pl.ds(out_base, nch * C)])

    @pl.when(cid == 0)
    def _():
        run(NCH0, sid * BPW0, sid * NCH0)

    @pl.when(cid == 1)
    def _():
        run(NCH1, NS * BPW0 + sid * BPW1, NS * NCH0 + sid * NCH1)


def kernel(nodes, to_neighs, features_table):
    del nodes
    b = to_neighs.shape[0]
    idx = jnp.pad(to_neighs.astype(jnp.int32), ((0, BP - b), (0, 0)))
    idx2d = idx.reshape(BP * S // (C * S), C * S)
    out = _mean_agg(idx2d, features_table)
    return out[:b]


# speedup vs baseline: 1.2940x; 1.0093x over previous
import functools

import jax
import jax.numpy as jnp
from jax import lax
from jax.experimental import pallas as pl
from jax.experimental.pallas import tpu as pltpu
from jax.experimental.pallas import tpu_sc as plsc

NC = 2
NS = 16
L = 16
S = 32
D = 128
C = 4
BP = 10240
NVREG = D // L
BPW0 = 608
BPW1 = 32
NCH0 = BPW0 // C
NCH1 = BPW1 // C


@functools.partial(
    pl.kernel,
    out_type=jax.ShapeDtypeStruct((BP, D), jnp.float32),
    mesh=plsc.VectorSubcoreMesh(
        core_axis_name="c", subcore_axis_name="s",
        num_cores=NC, num_subcores=NS),
    scratch_types=[
        pltpu.VMEM((NCH0, C * S), jnp.int32),
        pltpu.VMEM((C * S, D), jnp.float32),
        pltpu.VMEM((C * S, D), jnp.float32),
        pltpu.VMEM((BPW0, D), jnp.float32),
        pltpu.SemaphoreType.DMA,
        pltpu.SemaphoreType.DMA,
    ],
)
def _mean_agg(idx_hbm, table_hbm, out_hbm, idx_v, buf0, buf1, out_v,
              sem0, sem1):
    cid = lax.axis_index("c")
    sid = lax.axis_index("s")

    bufs = (buf0, buf1)
    sems = (sem0, sem1)

    def compute(c, buf):
        for r in range(C):
            def body(s_, carry):
                row = r * S + s_
                return tuple(
                    a + buf[row, pl.ds(v * L, L)]
                    for v, a in enumerate(carry))
            acc = lax.fori_loop(
                0, S, body,
                tuple(jnp.zeros((L,), jnp.float32) for _ in range(NVREG)))
            orow = c * C + r
            for v in range(NVREG):
                out_v[orow, pl.ds(v * L, L)] = acc[v] * (1.0 / S)

    def run(nch, out_base, idx_base):
        pltpu.sync_copy(idx_hbm.at[pl.ds(idx_base, nch)],
                        idx_v.at[pl.ds(0, nch)])
        pltpu.async_copy(table_hbm.at[idx_v.at[0]], buf0, sem0)
        pltpu.async_copy(table_hbm.at[idx_v.at[1]], buf1, sem1)

        def outer(g, carry):
            for b in range(2):
                c = g * 2 + b
                pltpu.make_async_copy(
                    table_hbm.at[idx_v.at[0]], bufs[b], sems[b]).wait()
                compute(c, bufs[b])

                @pl.when(c + 2 < nch)
                def _():
                    pltpu.async_copy(
                        table_hbm.at[idx_v.at[c + 2]], bufs[b], sems[b])
            return carry

        lax.fori_loop(0, nch // 2, outer, 0)
        pltpu.sync_copy(out_v.at[pl.ds(0, nch * C)],
                        out_hbm.at[pl.ds(out_base, nch * C)])

    @pl.when(cid == 0)
    def _():
        run(NCH0, sid * BPW0, sid * NCH0)

    @pl.when(cid == 1)
    def _():
        run(NCH1, NS * BPW0 + sid * BPW1, NS * NCH0 + sid * NCH1)


def kernel(nodes, to_neighs, features_table):
    del nodes
    b = to_neighs.shape[0]
    idx = jnp.pad(to_neighs.astype(jnp.int32), ((0, BP - b), (0, 0)))
    idx2d = idx.reshape(BP * S // (C * S), C * S)
    out = _mean_agg(idx2d, features_table)
    return out[:b]
